# baseline (speedup 1.0000x reference)
import functools

import numpy as np
import jax
import jax.numpy as jnp
from jax.experimental import pallas as pl
from jax.experimental.pallas import tpu as pltpu


def _up_masks(Wu):
    Wd = 2 * Wu
    m = np.zeros((2, Wd, Wu), np.float32)
    for w in range(Wu):
        for dj in range(2):
            m[dj, 2 * w + dj, w] = 1.0
    return m


def _band_masks(Wd):
    m = np.zeros((3, Wd, Wd), np.float32)
    for j in range(Wd):
        for dx in range(3):
            i = j + dx - 1
            if 0 <= i < Wd:
                m[dx, j, i] = 1.0
    return m


def _up_pair_mats(wt, Wu):
    Cin, Cout = wt.shape[0], wt.shape[1]
    Wd = 2 * Wu
    masks = _up_masks(Wu)
    wtp = jnp.transpose(wt, (2, 1, 0, 3)).astype(jnp.float32)
    m = sum(wtp[:, :, None, :, None, dj] * masks[dj][None, None, :, None, :]
            for dj in range(2))
    return m.reshape(2, Cout * Wd, Cin * Wu)


def _band_mats(w_oihw, Wd):
    Cout, Cin = w_oihw.shape[0], w_oihw.shape[1]
    masks = _band_masks(Wd)
    wp = jnp.transpose(w_oihw, (2, 0, 1, 3)).astype(jnp.float32)
    m = sum(wp[:, :, None, :, None, dx] * masks[dx][None, None, :, None, :]
            for dx in range(3))
    return m.reshape(3, Cout * Wd, Cin * Wd)


def _dot(a, b):
    return jnp.dot(a, b, preferred_element_type=jnp.float32)


def _cshift_m1(x):
    return jnp.concatenate([jnp.zeros_like(x[:, :1]), x[:, :-1]], axis=1)


def _cshift_p1(x):
    return jnp.concatenate([x[:, 1:], jnp.zeros_like(x[:, :1])], axis=1)


def _fused_kernel(Hu, Wu, Cin, Cout,
                  fu_ref, fd_ref, mu_ref, w1u_ref, w1f_ref, w2_ref,
                  btc_ref, b1c_ref, b2c_ref, o_ref):
    bf16 = jnp.bfloat16
    Wd = 2 * Wu
    Nw = Cout * Wd

    fu = fu_ref[0].astype(bf16)
    xfu = jnp.transpose(fu, (0, 2, 1)).reshape(Cin * Wu, Hu)
    fde = fd_ref[0, :, :, 0, :].astype(bf16)
    fdo = fd_ref[0, :, :, 1, :].astype(bf16)
    fd_e = jnp.transpose(fde, (0, 2, 1)).reshape(Nw, Hu)
    fd_o = jnp.transpose(fdo, (0, 2, 1)).reshape(Nw, Hu)

    up_e = (_dot(mu_ref[0], xfu) + btc_ref[...]).astype(bf16)
    up_o = (_dot(mu_ref[1], xfu) + btc_ref[...]).astype(bf16)

    uo_m1, fo_m1 = _cshift_m1(up_o), _cshift_m1(fd_o)
    ue_p1, fe_p1 = _cshift_p1(up_e), _cshift_p1(fd_e)
    h1e = (_dot(w1u_ref[0], uo_m1) + _dot(w1f_ref[0], fo_m1)
           + _dot(w1u_ref[1], up_e) + _dot(w1f_ref[1], fd_e)
           + _dot(w1u_ref[2], up_o) + _dot(w1f_ref[2], fd_o))
    h1o = (_dot(w1u_ref[0], up_e) + _dot(w1f_ref[0], fd_e)
           + _dot(w1u_ref[1], up_o) + _dot(w1f_ref[1], fd_o)
           + _dot(w1u_ref[2], ue_p1) + _dot(w1f_ref[2], fe_p1))
    h1e = jnp.maximum(h1e + b1c_ref[...], 0.0).astype(bf16)
    h1o = jnp.maximum(h1o + b1c_ref[...], 0.0).astype(bf16)

    ho_m1 = _cshift_m1(h1o)
    he_p1 = _cshift_p1(h1e)
    oe = _dot(w2_ref[0], ho_m1) + _dot(w2_ref[1], h1e) + _dot(w2_ref[2], h1o)
    oo = _dot(w2_ref[0], h1e) + _dot(w2_ref[1], h1o) + _dot(w2_ref[2], he_p1)
    oe = jnp.maximum(oe + b2c_ref[...], 0.0)
    oo = jnp.maximum(oo + b2c_ref[...], 0.0)

    oe3 = jnp.transpose(oe.reshape(Cout, Wd, Hu), (0, 2, 1))
    oo3 = jnp.transpose(oo.reshape(Cout, Wd, Hu), (0, 2, 1))
    o_ref[0, :, :, 0, :] = oe3
    o_ref[0, :, :, 1, :] = oo3


def kernel(from_down, from_up, wt, bt, w1, b1, w2, b2):
    N, Cout, Hd, Wd = from_down.shape
    _, Cin, Hu, Wu = from_up.shape
    bf16 = jnp.bfloat16
    Ku = Cin * Wu
    Nw = Cout * Wd

    fd = from_down.reshape(N, Cout, Hu, 2, Wd)

    mu = _up_pair_mats(wt, Wu).astype(bf16)
    w1u = _band_mats(w1[:, :Cout], Wd).astype(bf16)
    w1f = _band_mats(w1[:, Cout:], Wd).astype(bf16)
    w2b = _band_mats(w2, Wd).astype(bf16)
    btc = jnp.repeat(bt.astype(jnp.float32), Wd).reshape(Nw, 1)
    b1c = jnp.repeat(b1.astype(jnp.float32), Wd).reshape(Nw, 1)
    b2c = jnp.repeat(b2.astype(jnp.float32), Wd).reshape(Nw, 1)

    out = pl.pallas_call(
        functools.partial(_fused_kernel, Hu, Wu, Cin, Cout),
        out_shape=jax.ShapeDtypeStruct((N, Cout, Hu, 2, Wd), jnp.float32),
        grid=(N,),
        in_specs=[
            pl.BlockSpec((1, Cin, Hu, Wu), lambda n: (n, 0, 0, 0)),
            pl.BlockSpec((1, Cout, Hu, 2, Wd), lambda n: (n, 0, 0, 0, 0)),
            pl.BlockSpec((2, Nw, Ku), lambda n: (0, 0, 0)),
            pl.BlockSpec((3, Nw, Nw), lambda n: (0, 0, 0)),
            pl.BlockSpec((3, Nw, Nw), lambda n: (0, 0, 0)),
            pl.BlockSpec((3, Nw, Nw), lambda n: (0, 0, 0)),
            pl.BlockSpec((Nw, 1), lambda n: (0, 0)),
            pl.BlockSpec((Nw, 1), lambda n: (0, 0)),
            pl.BlockSpec((Nw, 1), lambda n: (0, 0)),
        ],
        out_specs=pl.BlockSpec((1, Cout, Hu, 2, Wd), lambda n: (n, 0, 0, 0, 0)),
        compiler_params=pltpu.CompilerParams(
            dimension_semantics=("parallel",),
            vmem_limit_bytes=64 * 1024 * 1024,
        ),
    )(from_up, fd, mu, w1u, w1f, w2b, btc, b1c, b2c)

    return out.reshape(N, Cout, Hd, Wd)


# speedup vs baseline: 1.0191x; 1.0191x over previous
import functools

import numpy as np
import jax
import jax.numpy as jnp
from jax.experimental import pallas as pl
from jax.experimental.pallas import tpu as pltpu


def _up_masks(Wu):
    Wd = 2 * Wu
    m = np.zeros((2, Wd, Wu), np.float32)
    for w in range(Wu):
        for dj in range(2):
            m[dj, 2 * w + dj, w] = 1.0
    return m


def _band_masks(Wd):
    m = np.zeros((3, Wd, Wd), np.float32)
    for j in range(Wd):
        for dx in range(3):
            i = j + dx - 1
            if 0 <= i < Wd:
                m[dx, j, i] = 1.0
    return m


def _interleave_mats(H):
    s = np.zeros((2, H // 2, H), np.float32)
    for i in range(H // 2):
        s[0, i, 2 * i] = 1.0
        s[1, i, 2 * i + 1] = 1.0
    return s


def _up_pair_mats(wt, Wu):
    Cin, Cout = wt.shape[0], wt.shape[1]
    Wd = 2 * Wu
    masks = _up_masks(Wu)
    wtp = jnp.transpose(wt, (2, 1, 0, 3)).astype(jnp.float32)
    m = sum(wtp[:, :, None, :, None, dj] * masks[dj][None, None, :, None, :]
            for dj in range(2))
    return m.reshape(2, Cout * Wd, Cin * Wu)


def _band_mats(w_oihw, Wd):
    Cout, Cin = w_oihw.shape[0], w_oihw.shape[1]
    masks = _band_masks(Wd)
    wp = jnp.transpose(w_oihw, (2, 0, 1, 3)).astype(jnp.float32)
    m = sum(wp[:, :, None, :, None, dx] * masks[dx][None, None, :, None, :]
            for dx in range(3))
    return m.reshape(3, Cout * Wd, Cin * Wd)


def _dot(a, b):
    return jnp.dot(a, b, preferred_element_type=jnp.float32)


def _shl(x):
    return jnp.concatenate([jnp.zeros_like(x[:, :1]), x[:, :-1]], axis=1)


def _shr(x):
    return jnp.concatenate([x[:, 1:], jnp.zeros_like(x[:, :1])], axis=1)


def _fused_kernel(Hu, Wu, Cin, Cout,
                  fu_ref, fd_ref, mu_ref, w1u_ref, w1f_ref, w2_ref,
                  si_ref, btb_ref, b1b_ref, b2b_ref, o_ref):
    bf16 = jnp.bfloat16
    Wd = 2 * Wu
    Hd = 2 * Hu
    Nw = Cout * Wd

    fu = fu_ref[0].astype(bf16)
    xfu = jnp.transpose(fu, (0, 2, 1)).reshape(Cin * Wu, Hu)
    fd = fd_ref[0].astype(bf16)
    fdp = jnp.transpose(fd, (0, 2, 1)).reshape(Nw, Hd)

    up_e = _dot(mu_ref[0], xfu).astype(bf16)
    up_o = _dot(mu_ref[1], xfu).astype(bf16)
    up = (_dot(up_e, si_ref[0]) + _dot(up_o, si_ref[1])
          + btb_ref[...]).astype(bf16)

    um1, up1 = _shl(up), _shr(up)
    fm1, fp1 = _shl(fdp), _shr(fdp)
    h1 = (_dot(w1u_ref[0], um1) + _dot(w1f_ref[0], fm1)
          + _dot(w1u_ref[1], up) + _dot(w1f_ref[1], fdp)
          + _dot(w1u_ref[2], up1) + _dot(w1f_ref[2], fp1))
    h1 = jnp.maximum(h1 + b1b_ref[...], 0.0).astype(bf16)

    h2 = (_dot(w2_ref[0], _shl(h1)) + _dot(w2_ref[1], h1)
          + _dot(w2_ref[2], _shr(h1)))
    h2 = jnp.maximum(h2 + b2b_ref[...], 0.0)

    o_ref[0] = jnp.transpose(h2.reshape(Cout, Wd, Hd), (0, 2, 1))


def kernel(from_down, from_up, wt, bt, w1, b1, w2, b2):
    N, Cout, Hd, Wd = from_down.shape
    _, Cin, Hu, Wu = from_up.shape
    bf16 = jnp.bfloat16
    Ku = Cin * Wu
    Nw = Cout * Wd

    mu = _up_pair_mats(wt, Wu).astype(bf16)
    w1u = _band_mats(w1[:, :Cout], Wd).astype(bf16)
    w1f = _band_mats(w1[:, Cout:], Wd).astype(bf16)
    w2b = _band_mats(w2, Wd).astype(bf16)
    si = jnp.asarray(_interleave_mats(Hd), dtype=bf16)
    btb = jnp.broadcast_to(
        jnp.repeat(bt.astype(jnp.float32), Wd)[:, None], (Nw, Hd))
    b1b = jnp.broadcast_to(
        jnp.repeat(b1.astype(jnp.float32), Wd)[:, None], (Nw, Hd))
    b2b = jnp.broadcast_to(
        jnp.repeat(b2.astype(jnp.float32), Wd)[:, None], (Nw, Hd))

    return pl.pallas_call(
        functools.partial(_fused_kernel, Hu, Wu, Cin, Cout),
        out_shape=jax.ShapeDtypeStruct((N, Cout, Hd, Wd), jnp.float32),
        grid=(N,),
        in_specs=[
            pl.BlockSpec((1, Cin, Hu, Wu), lambda n: (n, 0, 0, 0)),
            pl.BlockSpec((1, Cout, Hd, Wd), lambda n: (n, 0, 0, 0)),
            pl.BlockSpec((2, Nw, Ku), lambda n: (0, 0, 0)),
            pl.BlockSpec((3, Nw, Nw), lambda n: (0, 0, 0)),
            pl.BlockSpec((3, Nw, Nw), lambda n: (0, 0, 0)),
            pl.BlockSpec((3, Nw, Nw), lambda n: (0, 0, 0)),
            pl.BlockSpec((2, Hu, Hd), lambda n: (0, 0, 0)),
            pl.BlockSpec((Nw, Hd), lambda n: (0, 0)),
            pl.BlockSpec((Nw, Hd), lambda n: (0, 0)),
            pl.BlockSpec((Nw, Hd), lambda n: (0, 0)),
        ],
        out_specs=pl.BlockSpec((1, Cout, Hd, Wd), lambda n: (n, 0, 0, 0)),
        compiler_params=pltpu.CompilerParams(
            dimension_semantics=("parallel",),
            vmem_limit_bytes=64 * 1024 * 1024,
        ),
    )(from_up, from_down, mu, w1u, w1f, w2b, si, btb, b1b, b2b)


# speedup vs baseline: 1.9248x; 1.8888x over previous
import functools

import numpy as np
import jax
import jax.numpy as jnp
from jax.experimental import pallas as pl
from jax.experimental.pallas import tpu as pltpu


def _band_masks(Wd):
    m = np.zeros((3, Wd, Wd), np.float32)
    for j in range(Wd):
        for dx in range(3):
            i = j + dx - 1
            if 0 <= i < Wd:
                m[dx, j, i] = 1.0
    return m


def _up_masks(Wu):
    Wd = 2 * Wu
    m = np.zeros((2, Wd, Wu), np.float32)
    for w in range(Wu):
        for dj in range(2):
            m[dj, 2 * w + dj, w] = 1.0
    return m


def _block_sel(C, W):
    s = np.zeros((C * W, C), np.float32)
    for r in range(C * W):
        s[r, r // W] = 1.0
    return s


def _interleave_mats(H):
    s = np.zeros((2, H // 2, H), np.float32)
    for i in range(H // 2):
        s[0, i, 2 * i] = 1.0
        s[1, i, 2 * i + 1] = 1.0
    return s


def _builder_kernel(Cout, Cin, Wu,
                    w1p_ref, w2p_ref, wtp_ref, ro_ref, rcT_ref, ruT_ref,
                    mb_ref, mup_ref, w1u_ref, w1f_ref, w2b_ref, mu_ref):
    bf16 = jnp.bfloat16
    f32 = jnp.float32
    ro = ro_ref[...]
    rcT = rcT_ref[...]
    ruT = ruT_ref[...]

    def big(wsmall, rT):
        t = jnp.dot(ro, wsmall, preferred_element_type=f32)
        return jnp.dot(t, rT, preferred_element_type=f32)

    for k in range(3):
        au = af = az = None
        for dx in range(3):
            wkx = w1p_ref[3 * k + dx]
            m = mb_ref[dx]
            tu = big(wkx[:, :Cout], rcT) * m
            tf = big(wkx[:, Cout:], rcT) * m
            t2 = big(w2p_ref[3 * k + dx], rcT) * m
            au = tu if au is None else au + tu
            af = tf if af is None else af + tf
            az = t2 if az is None else az + t2
        w1u_ref[k] = au.astype(bf16)
        w1f_ref[k] = af.astype(bf16)
        w2b_ref[k] = az.astype(bf16)

    for p in range(2):
        acc = None
        for dj in range(2):
            t = big(wtp_ref[2 * p + dj], ruT) * mup_ref[dj]
            acc = t if acc is None else acc + t
        mu_ref[p] = acc.astype(bf16)


def _build_weight_mats(w1, w2, wt, Wu):
    Cout = w2.shape[0]
    Wd = 2 * Wu
    Nw = Cout * Wd
    Ku = wt.shape[0] * Wu
    f32 = jnp.float32
    w1p = jnp.transpose(w1, (2, 3, 0, 1)).reshape(9, Cout, 2 * Cout)
    w2p = jnp.transpose(w2, (2, 3, 0, 1)).reshape(9, Cout, Cout)
    wtp = jnp.transpose(wt, (2, 3, 1, 0)).reshape(4, Cout, wt.shape[0])
    ro = _block_sel(Cout, Wd)
    rcT = _block_sel(Cout, Wd).T
    ruT = _block_sel(wt.shape[0], Wu).T
    mb = np.tile(_band_masks(Wd), (1, Cout, Cout))
    mup = np.tile(_up_masks(Wu), (1, Cout, wt.shape[0]))

    return pl.pallas_call(
        functools.partial(_builder_kernel, Cout, wt.shape[0], Wu),
        out_shape=(
            jax.ShapeDtypeStruct((3, Nw, Nw), jnp.bfloat16),
            jax.ShapeDtypeStruct((3, Nw, Nw), jnp.bfloat16),
            jax.ShapeDtypeStruct((3, Nw, Nw), jnp.bfloat16),
            jax.ShapeDtypeStruct((2, Nw, Ku), jnp.bfloat16),
        ),
    )(w1p.astype(f32), w2p.astype(f32), wtp.astype(f32),
      jnp.asarray(ro), jnp.asarray(rcT), jnp.asarray(ruT),
      jnp.asarray(mb), jnp.asarray(mup))


def _dot(a, b):
    return jnp.dot(a, b, preferred_element_type=jnp.float32)


def _shl(x):
    return jnp.concatenate([jnp.zeros_like(x[:, :1]), x[:, :-1]], axis=1)


def _shr(x):
    return jnp.concatenate([x[:, 1:], jnp.zeros_like(x[:, :1])], axis=1)


def _fused_kernel(Hu, Wu, Cin, Cout,
                  fu_ref, fd_ref, mu_ref, w1u_ref, w1f_ref, w2_ref,
                  si_ref, btb_ref, b1b_ref, b2b_ref, o_ref):
    bf16 = jnp.bfloat16
    Wd = 2 * Wu
    Hd = 2 * Hu
    Nw = Cout * Wd

    xfu = jnp.transpose(fu_ref[0], (0, 2, 1)).reshape(Cin * Wu, Hu)
    fdp = jnp.transpose(fd_ref[0], (0, 2, 1)).reshape(Nw, Hd)

    up_e = _dot(mu_ref[0], xfu).astype(bf16)
    up_o = _dot(mu_ref[1], xfu).astype(bf16)
    up = (_dot(up_e, si_ref[0]) + _dot(up_o, si_ref[1])
          + btb_ref[...]).astype(bf16)

    h1 = (_dot(w1u_ref[0], _shl(up)) + _dot(w1f_ref[0], _shl(fdp))
          + _dot(w1u_ref[1], up) + _dot(w1f_ref[1], fdp)
          + _dot(w1u_ref[2], _shr(up)) + _dot(w1f_ref[2], _shr(fdp)))
    h1 = jnp.maximum(h1 + b1b_ref[...], 0.0).astype(bf16)

    h2 = (_dot(w2_ref[0], _shl(h1)) + _dot(w2_ref[1], h1)
          + _dot(w2_ref[2], _shr(h1)))
    h2 = jnp.maximum(h2 + b2b_ref[...], 0.0)

    o_ref[0] = jnp.transpose(h2.reshape(Cout, Wd, Hd), (0, 2, 1)).astype(bf16)


def kernel(from_down, from_up, wt, bt, w1, b1, w2, b2):
    N, Cout, Hd, Wd = from_down.shape
    _, Cin, Hu, Wu = from_up.shape
    bf16 = jnp.bfloat16
    Ku = Cin * Wu
    Nw = Cout * Wd

    w1u, w1f, w2b, mu = _build_weight_mats(w1, w2, wt, Wu)
    si = jnp.asarray(_interleave_mats(Hd), dtype=bf16)
    btb = jnp.broadcast_to(
        jnp.repeat(bt.astype(jnp.float32), Wd)[:, None], (Nw, Hd))
    b1b = jnp.broadcast_to(
        jnp.repeat(b1.astype(jnp.float32), Wd)[:, None], (Nw, Hd))
    b2b = jnp.broadcast_to(
        jnp.repeat(b2.astype(jnp.float32), Wd)[:, None], (Nw, Hd))

    out = pl.pallas_call(
        functools.partial(_fused_kernel, Hu, Wu, Cin, Cout),
        out_shape=jax.ShapeDtypeStruct((N, Cout, Hd, Wd), bf16),
        grid=(N,),
        in_specs=[
            pl.BlockSpec((1, Cin, Hu, Wu), lambda n: (n, 0, 0, 0)),
            pl.BlockSpec((1, Cout, Hd, Wd), lambda n: (n, 0, 0, 0)),
            pl.BlockSpec((2, Nw, Ku), lambda n: (0, 0, 0)),
            pl.BlockSpec((3, Nw, Nw), lambda n: (0, 0, 0)),
            pl.BlockSpec((3, Nw, Nw), lambda n: (0, 0, 0)),
            pl.BlockSpec((3, Nw, Nw), lambda n: (0, 0, 0)),
            pl.BlockSpec((2, Hu, Hd), lambda n: (0, 0, 0)),
            pl.BlockSpec((Nw, Hd), lambda n: (0, 0)),
            pl.BlockSpec((Nw, Hd), lambda n: (0, 0)),
            pl.BlockSpec((Nw, Hd), lambda n: (0, 0)),
        ],
        out_specs=pl.BlockSpec((1, Cout, Hd, Wd), lambda n: (n, 0, 0, 0)),
        compiler_params=pltpu.CompilerParams(
            dimension_semantics=("parallel",),
            vmem_limit_bytes=64 * 1024 * 1024,
        ),
    )(from_up.astype(bf16), from_down.astype(bf16),
      mu, w1u, w1f, w2b, si, btb, b1b, b2b)

    return out.astype(jnp.float32)
